# baseline (device time: 12252 ns/iter reference)
import jax
import jax.numpy as jnp
from jax import lax
from jax.experimental import pallas as pl
from jax.experimental.pallas import tpu as pltpu

N_DEV = 4
E_LOCAL = 4
E_TOT = 16
N_TOK = 512
D_IN = 256
D_OUT = 512
CAP = 25
CHUNK = N_TOK // N_DEV
SLOTS = CAP
G = E_LOCAL * SLOTS


def kernel(x, router_W, route_idx, expert_W):
    del router_W

    def body(x_hbm, idx_hbm, ew_hbm, out_ref,
             x_ref, idx_ref, ew_ref, keep_ref, ranks_ref, ygall_ref,
             load_sems, send_sems, recv_sems, ready_sems):
        p = lax.axis_index("i")

        idx_dma = pltpu.make_async_copy(idx_hbm, idx_ref, load_sems.at[0])
        x_dma = pltpu.make_async_copy(x_hbm, x_ref, load_sems.at[1])
        ew_dma = pltpu.make_async_copy(ew_hbm, ew_ref, load_sems.at[2])
        idx_dma.start()
        x_dma.start()
        ew_dma.start()

        for d in range(1, N_DEV):
            pl.semaphore_signal(
                ready_sems.at[p], inc=1,
                device_id=((p + d) % N_DEV,),
                device_id_type=pl.DeviceIdType.MESH,
            )
        barrier = pltpu.get_barrier_semaphore()
        pl.semaphore_signal(barrier, inc=1)
        pl.semaphore_wait(barrier, 1)

        idx_dma.wait()
        idx = idx_ref[:, :]
        ecols = lax.broadcasted_iota(jnp.int32, (N_TOK, E_TOT), 1)
        ind = (idx == ecols).astype(jnp.float32)
        row = lax.broadcasted_iota(jnp.int32, (N_TOK, N_TOK), 0)
        col = lax.broadcasted_iota(jnp.int32, (N_TOK, N_TOK), 1)
        tri = (col < row).astype(jnp.bfloat16)
        ranks = jnp.dot(tri, ind.astype(jnp.bfloat16),
                        preferred_element_type=jnp.float32)
        keep = ind * (ranks < CAP).astype(jnp.float32)
        keep_ref[:, :] = keep
        ranks_ref[:, :] = ranks

        myrep = (lax.broadcasted_iota(jnp.int32, (E_TOT, G), 0)
                 == p * E_LOCAL
                 + lax.broadcasted_iota(jnp.int32, (E_TOT, G), 1) // SLOTS
                 ).astype(jnp.float32)
        keep_rep = jnp.dot(keep, myrep, preferred_element_type=jnp.float32)
        ranks_rep = jnp.dot(ranks, myrep, preferred_element_type=jnp.float32)
        rmod = (lax.broadcasted_iota(jnp.int32, (N_TOK, G), 1)
                % SLOTS).astype(jnp.float32)
        pm = (keep_rep * (ranks_rep == rmod).astype(jnp.float32)
              ).astype(jnp.bfloat16)

        kc = keep_ref[pl.ds(p * CHUNK, CHUNK), :]
        rc = ranks_ref[pl.ds(p * CHUNK, CHUNK), :]
        rmod2 = (lax.broadcasted_iota(jnp.int32, (CHUNK, G), 1)
                 % SLOTS).astype(jnp.float32)
        pcs = []
        for d in range(N_DEV):
            r = (p + d) % N_DEV
            rep = (lax.broadcasted_iota(jnp.int32, (E_TOT, G), 0)
                   == r * E_LOCAL
                   + lax.broadcasted_iota(jnp.int32, (E_TOT, G), 1) // SLOTS
                   ).astype(jnp.float32)
            kcr = jnp.dot(kc, rep, preferred_element_type=jnp.float32)
            rcr = jnp.dot(rc, rep, preferred_element_type=jnp.float32)
            pcs.append((kcr * (rcr == rmod2).astype(jnp.float32)
                        ).astype(jnp.bfloat16))

        x_dma.wait()
        xg = lax.dot_general(pm, x_ref[:, :].astype(jnp.bfloat16),
                             (((0,), (0,)), ((), ())),
                             preferred_element_type=jnp.float32)
        ew_dma.wait()
        for j in range(E_LOCAL):
            ygall_ref[p, pl.ds(j * SLOTS, SLOTS), :] = jnp.dot(
                xg[j * SLOTS:(j + 1) * SLOTS, :], ew_ref[j],
                preferred_element_type=jnp.float32).astype(jnp.bfloat16)

        rdma_by_d = {}
        for d in (2, 1, 3):
            q = (p + d) % N_DEV
            rdma = pltpu.make_async_remote_copy(
                src_ref=ygall_ref.at[p],
                dst_ref=ygall_ref.at[p],
                send_sem=send_sems.at[d - 1],
                recv_sem=recv_sems.at[d - 1],
                device_id=(q,),
                device_id_type=pl.DeviceIdType.MESH,
            )
            pl.semaphore_wait(ready_sems.at[q], 1)
            rdma.start()
            rdma_by_d[d] = rdma

        acc = jnp.dot(pcs[0], ygall_ref[p],
                      preferred_element_type=jnp.float32)
        for d in (1, 3, 2):
            rdma_by_d[d].wait_recv()
            acc = acc + jnp.dot(pcs[d], ygall_ref[(p + d) % N_DEV],
                                preferred_element_type=jnp.float32)
        out_ref[:, :] = acc
        for d in (2, 1, 3):
            rdma_by_d[d].wait_send()

    return pl.pallas_call(
        body,
        out_shape=jax.ShapeDtypeStruct((CHUNK, D_OUT), jnp.float32),
        in_specs=[
            pl.BlockSpec(memory_space=pl.ANY),
            pl.BlockSpec(memory_space=pl.ANY),
            pl.BlockSpec(memory_space=pl.ANY),
        ],
        out_specs=pl.BlockSpec(memory_space=pltpu.VMEM),
        scratch_shapes=[
            pltpu.VMEM((N_TOK, D_IN), jnp.float32),
            pltpu.VMEM((N_TOK, 1), jnp.int32),
            pltpu.VMEM((E_LOCAL, D_IN, D_OUT), jnp.float32),
            pltpu.VMEM((N_TOK, E_TOT), jnp.float32),
            pltpu.VMEM((N_TOK, E_TOT), jnp.float32),
            pltpu.VMEM((N_DEV, G, D_OUT), jnp.bfloat16),
            pltpu.SemaphoreType.DMA((3,)),
            pltpu.SemaphoreType.DMA((N_DEV - 1,)),
            pltpu.SemaphoreType.DMA((N_DEV - 1,)),
            pltpu.SemaphoreType.REGULAR((N_DEV,)),
        ],
        compiler_params=pltpu.CompilerParams(collective_id=0),
    )(x, route_idx, expert_W)


# device time: 11537 ns/iter; 1.0620x vs baseline; 1.0620x over previous
import jax
import jax.numpy as jnp
from jax import lax
from jax.experimental import pallas as pl
from jax.experimental.pallas import tpu as pltpu

N_DEV = 4
E_LOCAL = 4
E_TOT = 16
N_TOK = 512
D_IN = 256
D_OUT = 512
CAP = 25
CHUNK = N_TOK // N_DEV
SLOTS = CAP
G = E_LOCAL * SLOTS


def kernel(x, router_W, route_idx, expert_W):
    del router_W

    def body(x_ref, idx_ref, ew_ref, out_hbm,
             keept_ref, rankst_ref, ygall_ref, out_vmem,
             send_sems, recv_sems, out_sem, ready_sems):
        p = lax.axis_index("i")

        for d in range(1, N_DEV):
            pl.semaphore_signal(
                ready_sems.at[p], inc=1,
                device_id=((p + d) % N_DEV,),
                device_id_type=pl.DeviceIdType.MESH,
            )
        barrier = pltpu.get_barrier_semaphore()
        pl.semaphore_signal(barrier, inc=1)
        pl.semaphore_wait(barrier, 1)

        idxr = idx_ref[:, :]
        erow = lax.broadcasted_iota(jnp.int32, (E_TOT, N_TOK), 0)
        ind_t = (idxr == erow).astype(jnp.bfloat16)
        jj = lax.broadcasted_iota(jnp.int32, (N_TOK, N_TOK), 0)
        ii = lax.broadcasted_iota(jnp.int32, (N_TOK, N_TOK), 1)
        ut = (jj < ii).astype(jnp.bfloat16)
        ranks_t = jnp.dot(ind_t, ut,
                          preferred_element_type=jnp.float32)
        keep_t = (ind_t.astype(jnp.float32)
                  * (ranks_t < CAP).astype(jnp.float32))
        keept_ref[:, :] = keep_t
        rankst_ref[:, :] = ranks_t

        myrep = (lax.broadcasted_iota(jnp.int32, (E_TOT, G), 0)
                 == p * E_LOCAL
                 + lax.broadcasted_iota(jnp.int32, (E_TOT, G), 1) // SLOTS
                 ).astype(jnp.float32)
        tdot = (((0,), (0,)), ((), ()))
        keep_rep = lax.dot_general(keep_t, myrep, tdot,
                                   preferred_element_type=jnp.float32)
        ranks_rep = lax.dot_general(ranks_t, myrep, tdot,
                                    preferred_element_type=jnp.float32)
        rmod = (lax.broadcasted_iota(jnp.int32, (N_TOK, G), 1)
                % SLOTS).astype(jnp.float32)
        pm = (keep_rep * (ranks_rep == rmod).astype(jnp.float32)
              ).astype(jnp.bfloat16)

        kc = keept_ref[:, pl.ds(p * CHUNK, CHUNK)]
        rc = rankst_ref[:, pl.ds(p * CHUNK, CHUNK)]
        rmod2 = (lax.broadcasted_iota(jnp.int32, (CHUNK, G), 1)
                 % SLOTS).astype(jnp.float32)
        pcs = []
        for d in range(N_DEV):
            r = (p + d) % N_DEV
            rep = (lax.broadcasted_iota(jnp.int32, (E_TOT, G), 0)
                   == r * E_LOCAL
                   + lax.broadcasted_iota(jnp.int32, (E_TOT, G), 1) // SLOTS
                   ).astype(jnp.float32)
            kcr = lax.dot_general(kc, rep, tdot,
                                  preferred_element_type=jnp.float32)
            rcr = lax.dot_general(rc, rep, tdot,
                                  preferred_element_type=jnp.float32)
            pcs.append((kcr * (rcr == rmod2).astype(jnp.float32)
                        ).astype(jnp.bfloat16))

        xg = lax.dot_general(pm, x_ref[:, :].astype(jnp.bfloat16), tdot,
                             preferred_element_type=jnp.float32)
        for j in range(E_LOCAL):
            ygall_ref[p, pl.ds(j * SLOTS, SLOTS), :] = jnp.dot(
                xg[j * SLOTS:(j + 1) * SLOTS, :], ew_ref[j],
                preferred_element_type=jnp.float32).astype(jnp.bfloat16)

        rdma_by_d = {}
        for d in (2, 1, 3):
            q = (p + d) % N_DEV
            rdma = pltpu.make_async_remote_copy(
                src_ref=ygall_ref.at[p],
                dst_ref=ygall_ref.at[p],
                send_sem=send_sems.at[d - 1],
                recv_sem=recv_sems.at[d - 1],
                device_id=(q,),
                device_id_type=pl.DeviceIdType.MESH,
            )
            pl.semaphore_wait(ready_sems.at[q], 1)
            rdma.start()
            rdma_by_d[d] = rdma

        acc = jnp.dot(pcs[0], ygall_ref[p],
                      preferred_element_type=jnp.float32)
        for d in (1, 3, 2):
            rdma_by_d[d].wait_recv()
            acc = acc + jnp.dot(pcs[d], ygall_ref[(p + d) % N_DEV],
                                preferred_element_type=jnp.float32)
        out_vmem[:, :] = acc
        out_dma = pltpu.make_async_copy(out_vmem, out_hbm, out_sem)
        out_dma.start()
        out_dma.wait()
        for d in (2, 1, 3):
            rdma_by_d[d].wait_send()

    idx_row = route_idx.reshape(1, N_TOK)
    return pl.pallas_call(
        body,
        out_shape=jax.ShapeDtypeStruct((CHUNK, D_OUT), jnp.float32),
        in_specs=[
            pl.BlockSpec(memory_space=pltpu.VMEM),
            pl.BlockSpec(memory_space=pltpu.VMEM),
            pl.BlockSpec(memory_space=pltpu.VMEM),
        ],
        out_specs=pl.BlockSpec(memory_space=pl.ANY),
        scratch_shapes=[
            pltpu.VMEM((E_TOT, N_TOK), jnp.float32),
            pltpu.VMEM((E_TOT, N_TOK), jnp.float32),
            pltpu.VMEM((N_DEV, G, D_OUT), jnp.bfloat16),
            pltpu.VMEM((CHUNK, D_OUT), jnp.float32),
            pltpu.SemaphoreType.DMA((N_DEV - 1,)),
            pltpu.SemaphoreType.DMA((N_DEV - 1,)),
            pltpu.SemaphoreType.DMA,
            pltpu.SemaphoreType.REGULAR((N_DEV,)),
        ],
        compiler_params=pltpu.CompilerParams(collective_id=0),
    )(x, idx_row, expert_W)
